# Initial kernel scaffold; baseline (speedup 1.0000x reference)
import jax
import jax.numpy as jnp
from jax.experimental import pallas as pl


def kernel(x, edge_index, edge_values, x0, W_weight, W_bias):
    raise NotImplementedError("write your pallas kernel here")


# speedup vs baseline: 8.6505x; 8.6505x over previous
import functools

import jax
import jax.numpy as jnp
from jax import lax
from jax.experimental import pallas as pl
from jax.experimental.pallas import tpu as pltpu
from jax.experimental.pallas import tpu_sc as plsc

N = 10000
N_PAD = 10240
E = 320000
D = 128

NC = 2
NS = 16
NW = NC * NS

CHUNK = 128
E_PAD = NW * 80 * CHUNK
NCHUNK = E_PAD // CHUNK
CPT = NCHUNK // NW
ROWS_PER_TILE = N_PAD // NS

_MESH = plsc.VectorSubcoreMesh(core_axis_name="c", subcore_axis_name="s")
_SC_PARAMS = pltpu.CompilerParams(needs_layout_passes=False)


@functools.partial(
    pl.kernel,
    out_type=(
        jax.ShapeDtypeStruct((N_PAD,), jnp.float32),
        jax.ShapeDtypeStruct((N_PAD,), jnp.float32),
    ),
    mesh=_MESH,
    scratch_types=[
        pltpu.VMEM((CPT, CHUNK), jnp.int32),
        pltpu.VMEM((CHUNK,), jnp.float32),
        pltpu.VMEM((ROWS_PER_TILE,), jnp.float32),
        pltpu.VMEM_SHARED((N_PAD,), jnp.float32),
    ],
    compiler_params=_SC_PARAMS,
)
def _degree_kernel(col_hbm, d0_hbm, d1_hbm, col_v, ones_v, zeros_v, d_sh):
    cid = lax.axis_index("c")
    sid = lax.axis_index("s")
    wid = sid * NC + cid

    def fill_zeros(i, _):
        zeros_v[pl.ds(i * 16, 16)] = jnp.zeros((16,), jnp.float32)
        return 0

    lax.fori_loop(0, ROWS_PER_TILE // 16, fill_zeros, 0)
    for k in range(CHUNK // 16):
        ones_v[pl.ds(k * 16, 16)] = jnp.ones((16,), jnp.float32)

    pltpu.sync_copy(zeros_v, d_sh.at[pl.ds(sid * ROWS_PER_TILE, ROWS_PER_TILE)])
    plsc.subcore_barrier()

    pltpu.sync_copy(col_hbm.at[pl.ds(wid * CPT, CPT)], col_v)

    def body(j, _):
        pltpu.sync_copy(ones_v, d_sh.at[col_v.at[j]], add=True)
        return 0

    lax.fori_loop(0, CPT, body, 0)
    plsc.subcore_barrier()

    sl = pl.ds(sid * ROWS_PER_TILE, ROWS_PER_TILE)

    @pl.when(cid == 0)
    def _():
        pltpu.sync_copy(d_sh.at[sl], d0_hbm.at[sl])

    @pl.when(cid == 1)
    def _():
        pltpu.sync_copy(d_sh.at[sl], d1_hbm.at[sl])


def _rsqrt_body(d0_ref, d1_ref, r_ref):
    d = d0_ref[...] + d1_ref[...]
    r_ref[...] = jnp.where(d > 0.0, lax.rsqrt(d), 0.0)


def _rsqrt(d0, d1):
    return pl.pallas_call(
        _rsqrt_body,
        out_shape=jax.ShapeDtypeStruct((8, N_PAD // 8), jnp.float32),
    )(d0.reshape(8, N_PAD // 8), d1.reshape(8, N_PAD // 8))


@functools.partial(
    pl.kernel,
    out_type=jax.ShapeDtypeStruct((NC, N_PAD, D), jnp.float32),
    mesh=_MESH,
    scratch_types=[
        pltpu.VMEM((CPT, CHUNK), jnp.int32),
        pltpu.VMEM((CPT, CHUNK), jnp.float32),
        pltpu.VMEM((N_PAD,), jnp.float32),
        pltpu.VMEM((CHUNK,), jnp.int32),
        pltpu.VMEM((CHUNK,), jnp.int32),
        pltpu.VMEM((CHUNK,), jnp.float32),
        pltpu.VMEM((CHUNK, D), jnp.float32),
        pltpu.VMEM_SHARED((N_PAD, D), jnp.float32),
        pltpu.SemaphoreType.DMA,
    ],
    compiler_params=_SC_PARAMS,
)
def _spmm_kernel(pk_hbm, ev_hbm, r_hbm, x_hbm, agg_hbm,
                 pk_v, ev_v, r_v, col_c, row_c, vals_v, rows_v, agg_sh, sem):
    cid = lax.axis_index("c")
    sid = lax.axis_index("s")
    wid = sid * NC + cid

    def zero_rows(t, _):
        rows_v[t // 8, pl.ds((t % 8) * 16, 16)] = jnp.zeros((16,), jnp.float32)
        return 0

    lax.fori_loop(0, CHUNK * 8, zero_rows, 0)
    for k in range(ROWS_PER_TILE // CHUNK):
        pltpu.sync_copy(
            rows_v, agg_sh.at[pl.ds(sid * ROWS_PER_TILE + k * CHUNK, CHUNK)]
        )
    plsc.subcore_barrier()

    pltpu.sync_copy(r_hbm, r_v)
    pltpu.sync_copy(pk_hbm.at[pl.ds(wid * CPT, CPT)], pk_v)
    pltpu.sync_copy(ev_hbm.at[pl.ds(wid * CPT, CPT)], ev_v)

    def chunk_body(j, _):
        for k in range(CHUNK // 16):
            sl = pl.ds(k * 16, 16)
            pk = pk_v[j, sl]
            cvec = jnp.bitwise_and(pk, 16383)
            rvec = lax.shift_right_logical(pk, 14)
            col_c[sl] = cvec
            row_c[sl] = rvec
            rc = plsc.load_gather(r_v, [cvec])
            rr = plsc.load_gather(r_v, [rvec])
            vals_v[sl] = ev_v[j, sl] * rc * rr
        pltpu.async_copy(x_hbm.at[col_c], rows_v, sem).wait()

        def edge_body(e, _):
            vb = plsc.load_gather(vals_v, [jnp.full((16,), e, jnp.int32)])
            for q in range(8):
                s2 = pl.ds(q * 16, 16)
                rows_v[e, s2] = rows_v[e, s2] * vb
            return 0

        lax.fori_loop(0, CHUNK, edge_body, 0)
        pltpu.sync_copy(rows_v, agg_sh.at[row_c], add=True)
        return 0

    lax.fori_loop(0, CPT, chunk_body, 0)
    plsc.subcore_barrier()
    pltpu.sync_copy(
        agg_sh.at[pl.ds(sid * ROWS_PER_TILE, ROWS_PER_TILE)],
        agg_hbm.at[cid, pl.ds(sid * ROWS_PER_TILE, ROWS_PER_TILE)],
    )


def _mm_body(a0_ref, a1_ref, wt_ref, b_ref, o_ref):
    a = a0_ref[...] + a1_ref[...]
    o_ref[...] = (
        jnp.dot(a, wt_ref[...], preferred_element_type=jnp.float32) + b_ref[...]
    )


def _matmul(a0, a1, wt, bias2d):
    mb = 1024
    return pl.pallas_call(
        _mm_body,
        grid=(N_PAD // mb,),
        in_specs=[
            pl.BlockSpec((mb, D), lambda i: (i, 0)),
            pl.BlockSpec((mb, D), lambda i: (i, 0)),
            pl.BlockSpec((D, D), lambda i: (0, 0)),
            pl.BlockSpec((1, D), lambda i: (0, 0)),
        ],
        out_specs=pl.BlockSpec((mb, D), lambda i: (i, 0)),
        out_shape=jax.ShapeDtypeStruct((N_PAD, D), jnp.float32),
    )(a0, a1, wt, bias2d)


def kernel(x, edge_index, edge_values, x0, W_weight, W_bias):
    pad = E_PAD - E
    pad_idx = jnp.full((pad,), N, jnp.int32)
    row_p = jnp.concatenate([edge_index[0], pad_idx])
    col_p = jnp.concatenate([edge_index[1], pad_idx])
    col2d = col_p.reshape(NCHUNK, CHUNK)
    pk2d = (row_p * 16384 + col_p).reshape(NCHUNK, CHUNK)
    ev2d = jnp.concatenate(
        [edge_values, jnp.zeros((pad,), jnp.float32)]
    ).reshape(NCHUNK, CHUNK)
    x_p = jnp.concatenate([x, jnp.zeros((N_PAD - N, D), jnp.float32)])

    d0, d1 = _degree_kernel(col2d)
    r = _rsqrt(d0, d1).reshape(N_PAD)
    agg2 = _spmm_kernel(pk2d, ev2d, r, x_p)
    out = _matmul(agg2[0], agg2[1], W_weight.T, W_bias.reshape(1, D))
    return out[:N]
